# baseline (speedup 1.0000x reference)
import functools

import jax
import jax.numpy as jnp
from jax import lax
from jax.experimental import pallas as pl
from jax.experimental.pallas import tpu as pltpu
from jax.experimental.pallas import tpu_sc as plsc

_B = 32
_H = 64
_W = 64
_N = _H * _W


def _sort_body(x_ref, ord_ref, sval_ref, key_ref, idx_ref):
    iota = lax.broadcasted_iota(jnp.int32, (_B, _N), 1)
    key_ref[...] = x_ref[...]
    idx_ref[...] = iota

    def stage(s, carry):
        k = jnp.int32(1) << s

        def cex(t, carry2):
            j = k >> (t + 1)
            key = key_ref[...]
            idx = idx_ref[...]
            low = (iota & j) == 0
            asc = (iota & k) == 0
            keep_small = jnp.logical_not(jnp.logical_xor(low, asc))
            sh_neg = jnp.int32(_N) - j
            pkey = jnp.where(low, pltpu.roll(key, sh_neg, 1),
                             pltpu.roll(key, j, 1))
            pidx = jnp.where(low, pltpu.roll(idx, sh_neg, 1),
                             pltpu.roll(idx, j, 1))
            swap = (keep_small & (key > pkey)) | (
                jnp.logical_not(keep_small) & (key < pkey))
            key_ref[...] = jnp.where(swap, pkey, key)
            idx_ref[...] = jnp.where(swap, pidx, idx)
            return carry2

        return lax.fori_loop(0, s, cex, carry)

    lax.fori_loop(1, 13, stage, jnp.int32(0))
    ord_ref[...] = idx_ref[...]
    sval_ref[...] = key_ref[...]


def _argsort_tc(flat):
    return pl.pallas_call(
        _sort_body,
        out_shape=(
            jax.ShapeDtypeStruct((_B, _N), jnp.int32),
            jax.ShapeDtypeStruct((_B, _N), jnp.float32),
        ),
        scratch_shapes=[
            pltpu.VMEM((_B, _N), jnp.float32),
            pltpu.VMEM((_B, _N), jnp.int32),
        ],
    )(flat)


def _uf_contrib(flat, order, svals):
    mesh = plsc.VectorSubcoreMesh(core_axis_name="c", subcore_axis_name="s")

    _NP = _N + 16

    @functools.partial(
        pl.kernel,
        mesh=mesh,
        out_type=jax.ShapeDtypeStruct((_B, 16), jnp.float32),
        compiler_params=pltpu.CompilerParams(needs_layout_passes=False),
        scratch_types=[
            pltpu.VMEM((_NP,), jnp.float32),
            pltpu.VMEM((_NP,), jnp.int32),
            pltpu.VMEM((_NP,), jnp.float32),
            pltpu.VMEM((_NP,), jnp.int32),
            pltpu.VMEM((_NP,), jnp.float32),
            pltpu.VMEM((16,), jnp.float32),
        ],
    )
    def uf(vals_hbm, order_hbm, svals_hbm, out_hbm, vals_v, order_v, svals_v,
           parent_v, len_v, row_v):
        b = lax.axis_index("s") * 2 + lax.axis_index("c")
        pltpu.sync_copy(vals_hbm.at[b], vals_v.at[pl.ds(0, _N)])
        pltpu.sync_copy(order_hbm.at[b], order_v.at[pl.ds(0, _N)])
        pltpu.sync_copy(svals_hbm.at[b], svals_v.at[pl.ds(0, _N)])

        lane = lax.iota(jnp.int32, 16)
        lane0 = lane == 0
        neg1 = jnp.full((16,), -1, jnp.int32)
        zeros16 = jnp.zeros((16,), jnp.float32)

        def sload(ref, i):
            return ref[pl.ds(i, 16)][0]

        def sstore(ref, i, v):
            plsc.store_scatter(ref, [jnp.full((16,), i, jnp.int32)],
                               jnp.full((16,), v), mask=lane0)

        def init_body(i, carry):
            parent_v[pl.ds(i * 16, 16)] = neg1
            len_v[pl.ds(i * 16, 16)] = zeros16
            return carry

        lax.fori_loop(0, _NP // 16, init_body, jnp.int32(0))

        roff = jnp.where(lane == 0, 1, jnp.where(lane == 1, -1, 0))
        coff = jnp.where(lane == 2, 1, jnp.where(lane == 3, -1, 0))
        poff = roff * _W + coff
        lane4 = lane < 4

        def chunk(i, cnt):
            ovec = order_v[pl.ds(i * 16, 16)]
            vvec = svals_v[pl.ds(i * 16, 16)]
            for kk in range(16):
                p = ovec[kk]
                vp = vvec[kk]
                sstore(parent_v, p, p)
                r0 = p // _W
                c0 = p % _W
                rvec = jnp.full((16,), r0, jnp.int32) + roff
                cvec = jnp.full((16,), c0, jnp.int32) + coff
                validv = (lane4 & (rvec >= 0) & (rvec < _H)
                          & (cvec >= 0) & (cvec < _W))
                pvec = jnp.full((16,), p, jnp.int32)
                qvec = pvec + jnp.where(validv, poff, 0)
                pqvec = plsc.load_gather(parent_v, [qvec])
                activev = (validv & (pqvec != -1)).astype(jnp.int32)

                def active_visit(carry):
                    rp, cnt, q, pq = carry
                    rq = lax.while_loop(lambda r: sload(parent_v, r) != r,
                                        lambda r: sload(parent_v, r), pq)
                    sstore(parent_v, q, rq)
                    merge = rq != rp
                    vrp = sload(vals_v, rp)
                    vrq = sload(vals_v, rq)
                    rp_elder = (vrp < vrq) | ((vrp == vrq) & (rp < rq))
                    young = jnp.where(rp_elder, rq, rp)
                    elder = jnp.where(rp_elder, rp, rq)
                    sstore(parent_v, young, elder)
                    length = jnp.where(merge, vp - jnp.maximum(vrp, vrq),
                                       jnp.float32(0.0))
                    sstore(len_v, cnt, length)
                    return (jnp.where(merge, elder, rp),
                            cnt + merge.astype(jnp.int32), q, pq)

                rp = p
                for d in range(4):
                    rp, cnt, _, _ = lax.cond(
                        activev[d] != 0, active_visit, lambda c: c,
                        (rp, cnt, qvec[d], pqvec[d]))
            return cnt

        lax.fori_loop(0, _N // 16, chunk, jnp.int32(0))

        def topk_body(i, top):
            chunk = len_v[pl.ds(i * 16, 16)]
            cdesc = lax.rev(lax.sort(chunk), (0,))
            return lax.sort(jnp.maximum(top, cdesc))

        top = lax.fori_loop(0, _N // 16, topk_body,
                            jnp.zeros((16,), jnp.float32))

        lane = lax.iota(jnp.int32, 16)
        sq = top * top
        contrib = jnp.where(lane == 15, 1.0 - sq,
                            jnp.where(lane >= 6, sq,
                                      jnp.zeros((16,), jnp.float32)))
        row_v[...] = contrib
        pltpu.sync_copy(row_v, out_hbm.at[b])

    return uf(flat, order, svals)


def kernel(data):
    assert data.shape == (_B, _H, _W), "check the shape!"
    flat = data.reshape(_B, _N)
    order, svals = _argsort_tc(flat)
    contrib = _uf_contrib(flat, order, svals)
    return jnp.sum(contrib)


# speedup vs baseline: 1.2881x; 1.2881x over previous
import functools

import jax
import jax.numpy as jnp
from jax import lax
from jax.experimental import pallas as pl
from jax.experimental.pallas import tpu as pltpu
from jax.experimental.pallas import tpu_sc as plsc

_B = 32
_H = 64
_W = 64
_N = _H * _W
_E = 2 * _N
_EREAL = _E - 2 * _W


def _edge_sort_body(x_ref, w_ref, eid_ref, key_ref, pay_ref):
    x = x_ref[...]
    colp = lax.broadcasted_iota(jnp.int32, (_B, _N), 1)
    inf = jnp.float32(jnp.inf)
    wh = jnp.where(colp % _W < _W - 1,
                   jnp.maximum(x, pltpu.roll(x, _N - 1, 1)), inf)
    wv = jnp.where(colp < _N - _W,
                   jnp.maximum(x, pltpu.roll(x, _N - _W, 1)), inf)
    key_ref[...] = jnp.concatenate([wh, wv], axis=1)
    iota = lax.broadcasted_iota(jnp.int32, (_B, _E), 1)
    pay_ref[...] = iota

    def stage(s, carry):
        k = jnp.int32(1) << s

        def cex(t, carry2):
            j = k >> (t + 1)
            key = key_ref[...]
            pay = pay_ref[...]
            low = (iota & j) == 0
            asc = (iota & k) == 0
            keep_small = jnp.logical_not(jnp.logical_xor(low, asc))
            sh_neg = jnp.int32(_E) - j
            pkey = jnp.where(low, pltpu.roll(key, sh_neg, 1),
                             pltpu.roll(key, j, 1))
            ppay = jnp.where(low, pltpu.roll(pay, sh_neg, 1),
                             pltpu.roll(pay, j, 1))
            swap = (keep_small & (key > pkey)) | (
                jnp.logical_not(keep_small) & (key < pkey))
            key_ref[...] = jnp.where(swap, pkey, key)
            pay_ref[...] = jnp.where(swap, ppay, pay)
            return carry2

        return lax.fori_loop(0, s, cex, carry)

    lax.fori_loop(1, 14, stage, jnp.int32(0))
    w_ref[...] = key_ref[...]
    eid_ref[...] = pay_ref[...]


def _edge_sort_tc(flat):
    return pl.pallas_call(
        _edge_sort_body,
        out_shape=(
            jax.ShapeDtypeStruct((_B, _E), jnp.float32),
            jax.ShapeDtypeStruct((_B, _E), jnp.int32),
        ),
        scratch_shapes=[
            pltpu.VMEM((_B, _E), jnp.float32),
            pltpu.VMEM((_B, _E), jnp.int32),
        ],
    )(flat)


def _uf_contrib(flat, w_sorted, eid_sorted):
    mesh = plsc.VectorSubcoreMesh(core_axis_name="c", subcore_axis_name="s")

    _NP = _N + 16
    _EP = _E + 16

    @functools.partial(
        pl.kernel,
        mesh=mesh,
        out_type=jax.ShapeDtypeStruct((_B, 16), jnp.float32),
        compiler_params=pltpu.CompilerParams(needs_layout_passes=False),
        scratch_types=[
            pltpu.VMEM((_NP,), jnp.float32),
            pltpu.VMEM((_EP,), jnp.float32),
            pltpu.VMEM((_EP,), jnp.int32),
            pltpu.VMEM((_NP,), jnp.int32),
            pltpu.VMEM((_NP,), jnp.float32),
            pltpu.VMEM((16,), jnp.float32),
        ],
    )
    def uf(vals_hbm, w_hbm, eid_hbm, out_hbm, vals_v, w_v, eid_v, parent_v,
           len_v, row_v):
        b = lax.axis_index("s") * 2 + lax.axis_index("c")
        pltpu.sync_copy(vals_hbm.at[b], vals_v.at[pl.ds(0, _N)])
        pltpu.sync_copy(w_hbm.at[b], w_v.at[pl.ds(0, _E)])
        pltpu.sync_copy(eid_hbm.at[b], eid_v.at[pl.ds(0, _E)])

        lane = lax.iota(jnp.int32, 16)
        lane0 = lane == 0
        zeros16 = jnp.zeros((16,), jnp.float32)

        def sload(ref, i):
            return ref[pl.ds(i, 16)][0]

        def sstore(ref, i, v):
            plsc.store_scatter(ref, [jnp.full((16,), i, jnp.int32)],
                               jnp.full((16,), v), mask=lane0)

        def init_body(i, carry):
            parent_v[pl.ds(i * 16, 16)] = lane + i * 16
            len_v[pl.ds(i * 16, 16)] = zeros16
            return carry

        lax.fori_loop(0, _NP // 16, init_body, jnp.int32(0))

        def find(i):
            return lax.while_loop(lambda r: sload(parent_v, r) != r,
                                  lambda r: sload(parent_v, r), i)

        def chunk(i, cnt):
            wvec = w_v[pl.ds(i * 16, 16)]
            evec = eid_v[pl.ds(i * 16, 16)]
            for kk in range(16):
                eid = evec[kk]
                w = wvec[kk]
                u = eid & (_N - 1)
                v = u + jnp.where(eid >= _N, _W, 1)
                ru = find(u)
                sstore(parent_v, u, ru)
                rv = find(v)
                sstore(parent_v, v, rv)

                def merge_fn(cnt):
                    vru = sload(vals_v, ru)
                    vrv = sload(vals_v, rv)
                    ru_elder = (vru < vrv) | ((vru == vrv) & (ru < rv))
                    young = jnp.where(ru_elder, rv, ru)
                    elder = jnp.where(ru_elder, ru, rv)
                    sstore(parent_v, young, elder)
                    sstore(len_v, cnt, w - jnp.maximum(vru, vrv))
                    return cnt + 1

                cnt = lax.cond(ru != rv, merge_fn, lambda c: c, cnt)
            return cnt

        lax.fori_loop(0, _EREAL // 16, chunk, jnp.int32(0))

        def topk_body(i, top):
            chunk16 = len_v[pl.ds(i * 16, 16)]
            cdesc = lax.rev(lax.sort(chunk16), (0,))
            return lax.sort(jnp.maximum(top, cdesc))

        top = lax.fori_loop(0, _NP // 16, topk_body,
                            jnp.zeros((16,), jnp.float32))

        sq = top * top
        contrib = jnp.where(lane == 15, 1.0 - sq,
                            jnp.where(lane >= 6, sq, zeros16))
        row_v[...] = contrib
        pltpu.sync_copy(row_v, out_hbm.at[b])

    return uf(flat, w_sorted, eid_sorted)


def kernel(data):
    assert data.shape == (_B, _H, _W), "check the shape!"
    flat = data.reshape(_B, _N)
    w_sorted, eid_sorted = _edge_sort_tc(flat)
    contrib = _uf_contrib(flat, w_sorted, eid_sorted)
    return jnp.sum(contrib)
